# baseline (speedup 1.0000x reference)
import jax
import jax.numpy as jnp
from jax.experimental import pallas as pl

_BV = 2048


def _mm_kernel(x_ref, w_ref, o_ref):
    o_ref[...] = jax.lax.dot_general(
        x_ref[...], w_ref[...],
        dimension_numbers=(((1,), (1,)), ((), ())),
        preferred_element_type=jnp.float32,
    )


def kernel(hidden_states, W):
    B, Q, H = hidden_states.shape
    V = W.shape[0]
    x = hidden_states.reshape(B * Q, H)
    out = pl.pallas_call(
        _mm_kernel,
        grid=(pl.cdiv(V, _BV),),
        in_specs=[
            pl.BlockSpec((B * Q, H), lambda i: (0, 0)),
            pl.BlockSpec((_BV, H), lambda i: (i, 0)),
        ],
        out_specs=pl.BlockSpec((B * Q, _BV), lambda i: (0, i)),
        out_shape=jax.ShapeDtypeStruct((B * Q, V), jnp.float32),
    )(x, W)
    return out.reshape(B, Q, V)
